# baseline (speedup 1.0000x reference)
import jax
import jax.numpy as jnp
from jax import lax
from jax.experimental import pallas as pl
from jax.experimental.pallas import tpu as pltpu
from jax.experimental.pallas import tpu_sc as plsc

_ANCHORS = (1.3221, 1.73145, 3.19275, 4.00944, 5.05587, 8.09892, 9.47112,
            4.84053, 11.2364, 10.0071)
_AW = _ANCHORS[0::2]
_AH = _ANCHORS[1::2]
_NA = 5
_NACT = 21
_NOBJ = 45
_C = 5 + _NACT + _NOBJ
_BS = 256
_G = 13
_S = _G * _G
_SIL = 0.6
_OBJ_SCALE = 5.0

_BB = 64
_SPW = 16
_NIDX = _SPW * _C
_NROW = 9
_NPAD = _NROW * 128


def _sig(x):
    return 1.0 / (1.0 + jnp.exp(-x))


def _sc_gather_body(pred_hbm, bbt_hbm, out_hbm, bb_v, idx_v, gath_v, sem):
    nc = 2
    wid = lax.axis_index("s") * nc + lax.axis_index("c")

    @pl.when(wid < _BS // _SPW)
    def _work():
        base = wid * _SPW
        for c in range(4):
            pltpu.sync_copy(bbt_hbm.at[c, pl.ds(base, _SPW)], bb_v.at[c])
        gx = bb_v[0] * float(_G)
        gy = bb_v[1] * float(_G)
        gw = bb_v[2] * float(_G)
        gh = bb_v[3] * float(_G)
        gi = jnp.clip(gx.astype(jnp.int32), 0, _G - 1)
        gj = jnp.clip(gy.astype(jnp.int32), 0, _G - 1)
        zero_i = jnp.zeros((_SPW,), jnp.int32)
        best = zero_i
        best_iou = jnp.zeros((_SPW,), jnp.float32) - 1.0
        for a in range(_NA):
            inter = jnp.minimum(gw, _AW[a]) * jnp.minimum(gh, _AH[a])
            union = gw * gh + _AW[a] * _AH[a] - inter
            iou = inter / jnp.maximum(union, 1e-10)
            upd = iou > best_iou
            best = jnp.where(upd, zero_i + a, best)
            best_iou = jnp.where(upd, iou, best_iou)
        lane = lax.broadcasted_iota(jnp.int32, (_SPW,), 0)
        fi0 = (base + lane) * (_NA * _C * _S) + best * (_C * _S) + gj * _G + gi
        for c in range(_C):
            pos = lane * _C + c
            plsc.store_scatter(idx_v,
                               [lax.div(pos, 128), lax.rem(pos, 128)],
                               fi0 + c * _S)
        ppos = _NIDX + lane
        plsc.store_scatter(idx_v,
                           [lax.div(ppos, 128), lax.rem(ppos, 128)], zero_i)
        copies = [
            pltpu.make_async_copy(pred_hbm.at[idx_v.at[j]],
                                  gath_v.at[pl.ds(j * 128, 128)], sem)
            for j in range(_NROW)
        ]
        for cp in copies:
            cp.start()
        for cp in copies:
            cp.wait()
        pltpu.sync_copy(gath_v.at[pl.ds(0, _NIDX)],
                        out_hbm.at[pl.ds(wid * _NIDX, _NIDX)])


def _make_sc_gather():
    return pl.kernel(
        _sc_gather_body,
        out_type=jax.ShapeDtypeStruct((_BS * _C,), jnp.float32),
        mesh=plsc.VectorSubcoreMesh(core_axis_name="c", subcore_axis_name="s"),
        scratch_types=[
            pltpu.VMEM((4, _SPW), jnp.float32),
            pltpu.VMEM((_NROW, 128), jnp.int32),
            pltpu.VMEM((_NPAD,), jnp.float32),
            pltpu.SemaphoreType.DMA,
        ],
        compiler_params=pltpu.CompilerParams(needs_layout_passes=False),
    )


def _dense_body(p5_ref, bb_ref, out_ref):
    @pl.when(pl.program_id(0) == 0)
    def _init():
        out_ref[...] = jnp.zeros_like(out_ref)

    x = p5_ref[...]
    bb = bb_ref[...]
    s = lax.broadcasted_iota(jnp.int32, (1, _S), 1)
    xi = (s % _G).astype(jnp.float32)
    yj = (s // _G).astype(jnp.float32)
    gx = bb[:, 0:1] * float(_G)
    gy = bb[:, 1:2] * float(_G)
    gw = bb[:, 2:3] * float(_G)
    gh = bb[:, 3:4] * float(_G)
    gx1 = gx - gw * 0.5
    gx2 = gx + gw * 0.5
    gy1 = gy - gh * 0.5
    gy2 = gy + gh * 0.5
    garea = gw * gh
    acc = jnp.zeros((), jnp.float32)
    for a in range(_NA):
        tx = _sig(x[:, 5 * a + 0, :])
        ty = _sig(x[:, 5 * a + 1, :])
        tw = x[:, 5 * a + 2, :]
        th = x[:, 5 * a + 3, :]
        pc = _sig(x[:, 5 * a + 4, :])
        px = tx + xi
        py = ty + yj
        pw = jnp.exp(tw) * _AW[a]
        ph = jnp.exp(th) * _AH[a]
        iw = jnp.maximum(jnp.minimum(px + pw * 0.5, gx2)
                         - jnp.maximum(px - pw * 0.5, gx1), 0.0)
        ih = jnp.maximum(jnp.minimum(py + ph * 0.5, gy2)
                         - jnp.maximum(py - ph * 0.5, gy1), 0.0)
        inter = iw * ih
        union = pw * ph + garea - inter
        iou = inter / jnp.maximum(union, 1e-10)
        mask = jnp.where(iou > _SIL, 0.0, 1.0)
        cell = ((tx - 0.5) ** 2 + (ty - 0.5) ** 2 + tw * tw + th * th
                + mask * pc * pc)
        acc = acc + jnp.sum(cell)
    out_ref[...] = out_ref[...] + (0.5 * acc).reshape(1, 1)


def _corr_body(g_ref, bb_ref, act_ref, obj_ref, out_ref):
    g = g_ref[...]
    bb = bb_ref[...]
    act = act_ref[...]
    obj = obj_ref[...]
    gx = bb[:, 0:1] * float(_G)
    gy = bb[:, 1:2] * float(_G)
    gw = bb[:, 2:3] * float(_G)
    gh = bb[:, 3:4] * float(_G)
    gi = jnp.clip(gx.astype(jnp.int32), 0, _G - 1)
    gj = jnp.clip(gy.astype(jnp.int32), 0, _G - 1)
    gif = gi.astype(jnp.float32)
    gjf = gj.astype(jnp.float32)
    best_iou = jnp.full_like(gx, -1.0)
    abw = jnp.zeros_like(gx)
    abh = jnp.zeros_like(gx)
    for a in range(_NA):
        inter = jnp.minimum(gw, _AW[a]) * jnp.minimum(gh, _AH[a])
        union = gw * gh + _AW[a] * _AH[a] - inter
        iou = inter / jnp.maximum(union, 1e-10)
        upd = iou > best_iou
        best_iou = jnp.where(upd, iou, best_iou)
        abw = jnp.where(upd, _AW[a], abw)
        abh = jnp.where(upd, _AH[a], abh)
    x0 = g[:, 0:1]
    x1 = g[:, 1:2]
    x2 = g[:, 2:3]
    x3 = g[:, 3:4]
    x4 = g[:, 4:5]
    tx = _sig(x0)
    ty = _sig(x1)
    pc = _sig(x4)
    coord_new = ((tx - (gx - gif)) ** 2 + (ty - (gy - gjf)) ** 2
                 + (x2 - jnp.log(gw / abw)) ** 2
                 + (x3 - jnp.log(gh / abh)) ** 2)
    coord_old = (tx - 0.5) ** 2 + (ty - 0.5) ** 2 + x2 * x2 + x3 * x3
    px = tx + gif
    py = ty + gjf
    pw = jnp.exp(x2) * abw
    ph = jnp.exp(x3) * abh
    iw = jnp.maximum(jnp.minimum(px + pw * 0.5, gx + gw * 0.5)
                     - jnp.maximum(px - pw * 0.5, gx - gw * 0.5), 0.0)
    ih = jnp.maximum(jnp.minimum(py + ph * 0.5, gy + gh * 0.5)
                     - jnp.maximum(py - ph * 0.5, gy - gh * 0.5), 0.0)
    inter = iw * ih
    union = pw * ph + gw * gh - inter
    iou_t = inter / jnp.maximum(union, 1e-10)
    conf_corr = (_OBJ_SCALE * (pc - iou_t) ** 2
                 - jnp.where(iou_t > _SIL, 0.0, 1.0) * pc * pc)
    la = g[:, 5:5 + _NACT]
    lo = g[:, 5 + _NACT:]
    ma = jnp.max(la, axis=1, keepdims=True)
    lse_a = ma + jnp.log(jnp.sum(jnp.exp(la - ma), axis=1, keepdims=True))
    sel_a = jnp.sum(jnp.where(
        lax.broadcasted_iota(jnp.int32, (_BS, _NACT), 1) == act, la, 0.0),
        axis=1, keepdims=True)
    mo = jnp.max(lo, axis=1, keepdims=True)
    lse_o = mo + jnp.log(jnp.sum(jnp.exp(lo - mo), axis=1, keepdims=True))
    sel_o = jnp.sum(jnp.where(
        lax.broadcasted_iota(jnp.int32, (_BS, _NOBJ), 1) == obj, lo, 0.0),
        axis=1, keepdims=True)
    total = jnp.sum(0.5 * (coord_new - coord_old + conf_corr)
                    - (sel_a - lse_a) - (sel_o - lse_o))
    out_ref[...] = total.reshape(1, 1)


def kernel(pred, bbox_gt, action_gt, obj_gt):
    pred = pred.astype(jnp.float32)
    pred_flat = pred.reshape(-1)
    bbt = bbox_gt.T
    gathered = _make_sc_gather()(pred_flat, bbt).reshape(_BS, _C)
    p5 = pred.reshape(_BS, _NA, _C, _S)[:, :, :5, :].reshape(_BS, _NA * 5, _S)
    dense = pl.pallas_call(
        _dense_body,
        grid=(_BS // _BB,),
        in_specs=[
            pl.BlockSpec((_BB, _NA * 5, _S), lambda i: (i, 0, 0)),
            pl.BlockSpec((_BB, 4), lambda i: (i, 0)),
        ],
        out_specs=pl.BlockSpec((1, 1), lambda i: (0, 0)),
        out_shape=jax.ShapeDtypeStruct((1, 1), jnp.float32),
    )(p5, bbox_gt)
    corr = pl.pallas_call(
        _corr_body,
        in_specs=[
            pl.BlockSpec((_BS, _C), lambda: (0, 0)),
            pl.BlockSpec((_BS, 4), lambda: (0, 0)),
            pl.BlockSpec((_BS, 1), lambda: (0, 0)),
            pl.BlockSpec((_BS, 1), lambda: (0, 0)),
        ],
        out_specs=pl.BlockSpec((1, 1), lambda: (0, 0)),
        out_shape=jax.ShapeDtypeStruct((1, 1), jnp.float32),
    )(gathered, bbox_gt,
      action_gt.reshape(_BS, 1).astype(jnp.int32),
      obj_gt.reshape(_BS, 1).astype(jnp.int32))
    return dense[0, 0] + corr[0, 0]
